# baseline (speedup 1.0000x reference)
import functools

import jax
import jax.numpy as jnp
from jax import lax
from jax.experimental import pallas as pl
from jax.experimental.pallas import tpu as pltpu
from jax.experimental.pallas import tpu_sc as plsc

N = 10000
R = 8
EMB = 128
E = 320000

NC, NS = 2, 16
NW = NC * NS
CH = 128
E_PAD = -(-E // (NW * CH * 8)) * (NW * CH * 8)
ROWS2D = E_PAD // CH
RA = ROWS2D // NS
RB0 = 104
RB1 = 56
NSEG = N * R
CPAD = 80128
CSL = CPAD // NS
APAD = 10112
ASL = APAD // NS


def _sc_body(t_hbm, fr_hbm, gsf_hbm, out_hbm,
             fra_a, fra_b, gsf, rows_a, rows_b, ones, c1_a, c1_b,
             zbuf, counts_sm, acc_sm,
             sem_l, sem_c, sem_g, sem_s, sem_f):
    c = lax.axis_index("c")
    s = lax.axis_index("s")
    zero16 = jnp.zeros((16,), jnp.float32)
    fra = (fra_a, fra_b)
    rows = (rows_a, rows_b)

    scope = jax.named_scope
    abase = s * ASL
    with scope("p0_zero"):
        def zrow(i, _):
            for k in range(8):
                rows_a[i, pl.ds(16 * k, 16)] = zero16
            return 0
        lax.fori_loop(0, CH, zrow, 0)

        def zlin(i, _):
            zbuf[pl.ds(i * 16, 16)] = zero16
            return 0
        lax.fori_loop(0, CH, zlin, 0)

        for k in range(8):
            ones[pl.ds(16 * k, 16)] = jnp.ones((16,), jnp.float32)

        hz = []
        for k in range(4):
            hz.append(pltpu.async_copy(
                rows_a, acc_sm.at[pl.ds(abase + k * CH, CH)], sem_l))
        hz.append(pltpu.async_copy(
            rows_a.at[pl.ds(0, ASL - 4 * CH)],
            acc_sm.at[pl.ds(abase + 4 * CH, ASL - 4 * CH)], sem_l))
        for k in range(2):
            hz.append(pltpu.async_copy(
                zbuf.at[pl.ds(0, 2048)],
                counts_sm.at[pl.ds(s * CSL + k * 2048, 2048)], sem_l))
        hz.append(pltpu.async_copy(
            zbuf.at[pl.ds(0, CSL - 4096)],
            counts_sm.at[pl.ds(s * CSL + 4096, CSL - 4096)], sem_l))
        for h in hz:
            h.wait()
        plsc.subcore_barrier()

    with scope("p1_count"):
        nga = RA // 4
        hl = [None, None]
        hl[0] = pltpu.async_copy(fr_hbm.at[pl.ds(s * RA, 4)], fra[0],
                                 sem_l)
        pend = []
        for b in range(nga):
            cur = b & 1
            hl[cur].wait()
            for h in pend:
                h.wait()
            if b + 1 < nga:
                hl[cur ^ 1] = pltpu.async_copy(
                    fr_hbm.at[pl.ds(s * RA + (b + 1) * 4, 4)],
                    fra[cur ^ 1], sem_l)
            pend = [pltpu.async_copy(ones, counts_sm.at[fra[cur].at[r]],
                                     sem_c, add=True)
                    for r in range(4)]
        for h in pend:
            h.wait()
        plsc.subcore_barrier()

    with scope("p2_recip"):
        for off, n in ((0, 2048), (2048, 2048), (4096, CSL - 4096)):
            pltpu.sync_copy(counts_sm.at[pl.ds(s * CSL + off, n)],
                            zbuf.at[pl.ds(0, n)])

            def rec(i, _):
                sl = pl.ds(i * 16, 16)
                zbuf[sl] = 1.0 / zbuf[sl]
                return 0
            lax.fori_loop(0, n // 16, rec, 0)
            pltpu.sync_copy(zbuf.at[pl.ds(0, n)],
                            counts_sm.at[pl.ds(s * CSL + off, n)])
        plsc.subcore_barrier()

    ebase = jnp.where(c == 0, s * RB0, NS * RB0 + s * RB1)
    nblk = jnp.where(c == 0, RB0 // 8, RB1 // 8)
    c1 = (c1_a, c1_b)
    lane = [jnp.full((16,), i, jnp.int32) for i in range(16)]

    def agg_blk(b, _):
        j0 = ebase + b * 8
        with scope("w_blk"):
            pltpu.sync_copy(gsf_hbm.at[pl.ds(j0 * 3, 24)], gsf)
        hg = [None, None]
        hs = [None, None]
        hc = [None, None]
        hc[0] = pltpu.async_copy(counts_sm.at[gsf.at[2]], c1[0], sem_c)
        hg[0] = pltpu.async_copy(t_hbm.at[gsf.at[0]], rows[0], sem_g)
        for r in range(8):
            cur = r & 1
            oth = cur ^ 1
            with scope("w_cnt"):
                hc[cur].wait()
            if r < 7:
                hc[oth] = pltpu.async_copy(
                    counts_sm.at[gsf.at[3 * r + 5]], c1[oth], sem_c)
                if r >= 1:
                    with scope("w_sct"):
                        hs[oth].wait()
                hg[oth] = pltpu.async_copy(t_hbm.at[gsf.at[3 * r + 3]],
                                           rows[oth], sem_g)
            with scope("w_gat"):
                hg[cur].wait()
            rbuf = rows[cur]
            vbuf = c1[cur]

            def scale(gq, _):
                cvec = vbuf[pl.ds(gq * 16, 16)]
                e0 = gq * 16
                for i in range(16):
                    vv = cvec.at[lane[i]].get(mode="promise_in_bounds")
                    for k in range(8):
                        sl = pl.ds(16 * k, 16)
                        rbuf[e0 + i, sl] = rbuf[e0 + i, sl] * vv
                return 0
            with scope("scale"):
                lax.fori_loop(0, CH // 16, scale, 0)
            hs[cur] = pltpu.async_copy(rows[cur],
                                       acc_sm.at[gsf.at[3 * r + 1]],
                                       sem_s, add=True)
        with scope("w_sct"):
            hs[0].wait()
            hs[1].wait()
        return 0
    with scope("p3_agg"):
        lax.fori_loop(0, nblk, agg_blk, 0)
        plsc.subcore_barrier()

    with scope("p4_out"):
        pltpu.sync_copy(acc_sm.at[pl.ds(abase, ASL)],
                        out_hbm.at[c, pl.ds(abase, ASL)])


_sc_agg = functools.partial(
    pl.kernel,
    out_type=jax.ShapeDtypeStruct((NC, APAD, EMB), jnp.float32),
    mesh=plsc.VectorSubcoreMesh(core_axis_name="c", subcore_axis_name="s"),
    scratch_types=[
        pltpu.VMEM((4, CH), jnp.int32),
        pltpu.VMEM((4, CH), jnp.int32),
        pltpu.VMEM((24, CH), jnp.int32),
        pltpu.VMEM((CH, EMB), jnp.float32),
        pltpu.VMEM((CH, EMB), jnp.float32),
        pltpu.VMEM((CH,), jnp.float32),
        pltpu.VMEM((CH,), jnp.float32),
        pltpu.VMEM((CH,), jnp.float32),
        pltpu.VMEM((16 * CH,), jnp.float32),
        pltpu.VMEM_SHARED((CPAD,), jnp.float32),
        pltpu.VMEM_SHARED((APAD, EMB), jnp.float32),
        pltpu.SemaphoreType.DMA,
        pltpu.SemaphoreType.DMA,
        pltpu.SemaphoreType.DMA,
        pltpu.SemaphoreType.DMA,
        pltpu.SemaphoreType.DMA,
    ],
)(_sc_body)


def _matmul_body(e_ref, w_ref, o_ref):
    o_ref[...] = lax.dot_general(
        e_ref[...], w_ref[0],
        dimension_numbers=(((1,), (1,)), ((), ())),
        preferred_element_type=jnp.float32)


def _finish_body(p0_ref, p1_ref, o_ref):
    o_ref[...] = jnp.maximum(p0_ref[0] + p1_ref[0], 0.0)


_BM = 1000


def kernel(embeddings, src, rel, dst, W):
    src = src.astype(jnp.int32)
    rel = rel.astype(jnp.int32)
    dst = dst.astype(jnp.int32)
    fr = src + N * rel
    g = rel * N + dst
    pad = E_PAD - E
    fr = jnp.concatenate([fr, jnp.full((pad,), NSEG, jnp.int32)])
    g = jnp.concatenate([g, jnp.zeros((pad,), jnp.int32)])
    sc = jnp.concatenate([src, jnp.full((pad,), N, jnp.int32)])
    fr = fr.reshape(ROWS2D, CH)
    g = g.reshape(ROWS2D, CH)
    sc = sc.reshape(ROWS2D, CH)
    gsf = jnp.stack([g, sc, fr], axis=1).reshape(ROWS2D * 3, CH)

    T = pl.pallas_call(
        _matmul_body,
        grid=(R, N // _BM),
        in_specs=[pl.BlockSpec((_BM, EMB), lambda r, i: (i, 0)),
                  pl.BlockSpec((1, EMB, EMB), lambda r, i: (r, 0, 0))],
        out_specs=pl.BlockSpec((_BM, EMB), lambda r, i: (r * (N // _BM) + i, 0)),
        out_shape=jax.ShapeDtypeStruct((NSEG, EMB), jnp.float32),
    )(embeddings, W)

    partials = _sc_agg(T, fr, gsf)

    out = pl.pallas_call(
        _finish_body,
        grid=(N // _BM,),
        in_specs=[pl.BlockSpec((1, _BM, EMB), lambda i: (0, i, 0)),
                  pl.BlockSpec((1, _BM, EMB), lambda i: (1, i, 0))],
        out_specs=pl.BlockSpec((_BM, EMB), lambda i: (i, 0)),
        out_shape=jax.ShapeDtypeStruct((N, EMB), jnp.float32),
    )(partials, partials)
    return out


# speedup vs baseline: 1.2234x; 1.2234x over previous
import functools

import jax
import jax.numpy as jnp
from jax import lax
from jax.experimental import pallas as pl
from jax.experimental.pallas import tpu as pltpu
from jax.experimental.pallas import tpu_sc as plsc

N = 10000
R = 8
EMB = 128
E = 320000

NC, NS = 2, 16
NW = NC * NS
CH = 128
E_PAD = -(-E // (NW * CH * 8)) * (NW * CH * 8)
ROWS2D = E_PAD // CH
RA = ROWS2D // NS
RB0 = 104
RB1 = 56
NSEG = N * R
CPAD = 80128
CSL = CPAD // NS
APAD = 10112
ASL = APAD // NS


def _sc_body(t_hbm, fr_hbm, g_hbm, s_hbm, out_hbm,
             fra_a, fra_b, g8, s8, fr8, rows_a, rows_b, ones, c1_a, c1_b,
             zbuf, counts_sm, acc_sm,
             sem_l, sem_c, sem_g, sem_s, sem_f):
    c = lax.axis_index("c")
    s = lax.axis_index("s")
    zero16 = jnp.zeros((16,), jnp.float32)
    fra = (fra_a, fra_b)
    rows = (rows_a, rows_b)

    scope = jax.named_scope
    abase = s * ASL
    with scope("p0_zero"):
        def zrow(i, _):
            for k in range(8):
                rows_a[i, pl.ds(16 * k, 16)] = zero16
            return 0
        lax.fori_loop(0, CH, zrow, 0)

        def zlin(i, _):
            zbuf[pl.ds(i * 16, 16)] = zero16
            return 0
        lax.fori_loop(0, CH, zlin, 0)

        for k in range(8):
            ones[pl.ds(16 * k, 16)] = jnp.ones((16,), jnp.float32)

        hz = []
        for k in range(4):
            hz.append(pltpu.async_copy(
                rows_a, acc_sm.at[pl.ds(abase + k * CH, CH)], sem_l))
        hz.append(pltpu.async_copy(
            rows_a.at[pl.ds(0, ASL - 4 * CH)],
            acc_sm.at[pl.ds(abase + 4 * CH, ASL - 4 * CH)], sem_l))
        for k in range(2):
            hz.append(pltpu.async_copy(
                zbuf.at[pl.ds(0, 2048)],
                counts_sm.at[pl.ds(s * CSL + k * 2048, 2048)], sem_l))
        hz.append(pltpu.async_copy(
            zbuf.at[pl.ds(0, CSL - 4096)],
            counts_sm.at[pl.ds(s * CSL + 4096, CSL - 4096)], sem_l))
        for h in hz:
            h.wait()
        plsc.subcore_barrier()

    with scope("p1_count"):
        nga = RA // 4
        hl = [None, None]
        hl[0] = pltpu.async_copy(fr_hbm.at[pl.ds(s * RA, 4)], fra[0],
                                 sem_l)
        pend = []
        for b in range(nga):
            cur = b & 1
            hl[cur].wait()
            for h in pend:
                h.wait()
            if b + 1 < nga:
                hl[cur ^ 1] = pltpu.async_copy(
                    fr_hbm.at[pl.ds(s * RA + (b + 1) * 4, 4)],
                    fra[cur ^ 1], sem_l)
            pend = [pltpu.async_copy(ones, counts_sm.at[fra[cur].at[r]],
                                     sem_c, add=True)
                    for r in range(4)]
        for h in pend:
            h.wait()
        plsc.subcore_barrier()

    with scope("p2_recip"):
        for off, n in ((0, 2048), (2048, 2048), (4096, CSL - 4096)):
            pltpu.sync_copy(counts_sm.at[pl.ds(s * CSL + off, n)],
                            zbuf.at[pl.ds(0, n)])

            def rec(i, _):
                sl = pl.ds(i * 16, 16)
                zbuf[sl] = 1.0 / zbuf[sl]
                return 0
            lax.fori_loop(0, n // 16, rec, 0)
            pltpu.sync_copy(zbuf.at[pl.ds(0, n)],
                            counts_sm.at[pl.ds(s * CSL + off, n)])
        plsc.subcore_barrier()

    ebase = jnp.where(c == 0, s * RB0, NS * RB0 + s * RB1)
    nblk = jnp.where(c == 0, RB0 // 8, RB1 // 8)
    c1 = (c1_a, c1_b)
    lane = [jnp.full((16,), i, jnp.int32) for i in range(16)]

    def agg_blk(b, _):
        j0 = ebase + b * 8
        with scope("w_blk"):
            pltpu.sync_copy(g_hbm.at[pl.ds(j0, 8)], g8)
            pltpu.sync_copy(s_hbm.at[pl.ds(j0, 8)], s8)
            pltpu.sync_copy(fr_hbm.at[pl.ds(j0, 8)], fr8)
        hg = [None, None]
        hs = [None, None]
        hc = [None, None]
        hc[0] = pltpu.async_copy(counts_sm.at[fr8.at[0]], c1[0], sem_c)
        hg[0] = pltpu.async_copy(t_hbm.at[g8.at[0]], rows[0], sem_g)
        for r in range(8):
            cur = r & 1
            oth = cur ^ 1
            with scope("w_cnt"):
                hc[cur].wait()
            if r < 7:
                hc[oth] = pltpu.async_copy(counts_sm.at[fr8.at[r + 1]],
                                           c1[oth], sem_c)
                if r >= 1:
                    with scope("w_sct"):
                        hs[oth].wait()
                hg[oth] = pltpu.async_copy(t_hbm.at[g8.at[r + 1]],
                                           rows[oth], sem_g)
            with scope("w_gat"):
                hg[cur].wait()
            rbuf = rows[cur]
            vbuf = c1[cur]

            def scale(gq, _):
                cvec = vbuf[pl.ds(gq * 16, 16)]
                e0 = gq * 16
                for i in range(16):
                    vv = cvec.at[lane[i]].get(mode="promise_in_bounds")
                    for k in range(8):
                        sl = pl.ds(16 * k, 16)
                        rbuf[e0 + i, sl] = rbuf[e0 + i, sl] * vv
                return 0
            with scope("scale"):
                lax.fori_loop(0, CH // 16, scale, 0)
            hs[cur] = pltpu.async_copy(rows[cur], acc_sm.at[s8.at[r]],
                                       sem_s, add=True)
        with scope("w_sct"):
            hs[0].wait()
            hs[1].wait()
        return 0
    with scope("p3_agg"):
        lax.fori_loop(0, nblk, agg_blk, 0)
        plsc.subcore_barrier()

    with scope("p4_out"):
        pltpu.sync_copy(acc_sm.at[pl.ds(abase, ASL)],
                        out_hbm.at[c, pl.ds(abase, ASL)])


_sc_agg = functools.partial(
    pl.kernel,
    out_type=jax.ShapeDtypeStruct((NC, APAD, EMB), jnp.float32),
    mesh=plsc.VectorSubcoreMesh(core_axis_name="c", subcore_axis_name="s"),
    scratch_types=[
        pltpu.VMEM((4, CH), jnp.int32),
        pltpu.VMEM((4, CH), jnp.int32),
        pltpu.VMEM((8, CH), jnp.int32),
        pltpu.VMEM((8, CH), jnp.int32),
        pltpu.VMEM((8, CH), jnp.int32),
        pltpu.VMEM((CH, EMB), jnp.float32),
        pltpu.VMEM((CH, EMB), jnp.float32),
        pltpu.VMEM((CH,), jnp.float32),
        pltpu.VMEM((CH,), jnp.float32),
        pltpu.VMEM((CH,), jnp.float32),
        pltpu.VMEM((16 * CH,), jnp.float32),
        pltpu.VMEM_SHARED((CPAD,), jnp.float32),
        pltpu.VMEM_SHARED((APAD, EMB), jnp.float32),
        pltpu.SemaphoreType.DMA,
        pltpu.SemaphoreType.DMA,
        pltpu.SemaphoreType.DMA,
        pltpu.SemaphoreType.DMA,
        pltpu.SemaphoreType.DMA,
    ],
)(_sc_body)


def _matmul_body(e_ref, w_ref, o_ref):
    o_ref[...] = lax.dot_general(
        e_ref[...], w_ref[0],
        dimension_numbers=(((1,), (1,)), ((), ())),
        preferred_element_type=jnp.float32)


def _finish_body(p0_ref, p1_ref, o_ref):
    o_ref[...] = jnp.maximum(p0_ref[0] + p1_ref[0], 0.0)


_BM = 1000


def kernel(embeddings, src, rel, dst, W):
    src = src.astype(jnp.int32)
    rel = rel.astype(jnp.int32)
    dst = dst.astype(jnp.int32)
    fr = src + N * rel
    g = rel * N + dst
    pad = E_PAD - E
    fr = jnp.concatenate([fr, jnp.full((pad,), NSEG, jnp.int32)])
    g = jnp.concatenate([g, jnp.zeros((pad,), jnp.int32)])
    sc = jnp.concatenate([src, jnp.full((pad,), N, jnp.int32)])
    fr = fr.reshape(ROWS2D, CH)
    g = g.reshape(ROWS2D, CH)
    sc = sc.reshape(ROWS2D, CH)

    T = pl.pallas_call(
        _matmul_body,
        grid=(R, N // _BM),
        in_specs=[pl.BlockSpec((_BM, EMB), lambda r, i: (i, 0)),
                  pl.BlockSpec((1, EMB, EMB), lambda r, i: (r, 0, 0))],
        out_specs=pl.BlockSpec((_BM, EMB), lambda r, i: (r * (N // _BM) + i, 0)),
        out_shape=jax.ShapeDtypeStruct((NSEG, EMB), jnp.float32),
    )(embeddings, W)

    partials = _sc_agg(T, fr, g, sc)

    out = pl.pallas_call(
        _finish_body,
        grid=(N // _BM,),
        in_specs=[pl.BlockSpec((1, _BM, EMB), lambda i: (0, i, 0)),
                  pl.BlockSpec((1, _BM, EMB), lambda i: (1, i, 0))],
        out_specs=pl.BlockSpec((_BM, EMB), lambda i: (i, 0)),
        out_shape=jax.ShapeDtypeStruct((N, EMB), jnp.float32),
    )(partials, partials)
    return out
